# baseline (speedup 1.0000x reference)
import functools

import jax
import jax.numpy as jnp
from jax import lax
from jax.experimental import pallas as pl
from jax.experimental.pallas import tpu as pltpu
from jax.experimental.pallas import tpu_sc as plsc

B = 2
L = 256
D_IN = 768
P = 256
Q = 1024
NQ = 2 * Q
ROWS = 2 * B * L


def _tc_tables(enc_ref, w_ref, bproj_ref, wattn_ref, c_ref, z_ref):
    enc = enc_ref[...]
    w = w_ref[...]
    x = lax.dot_general(enc, w, (((1,), (1,)), ((), ())),
                        preferred_element_type=jnp.float32)
    x = x + bproj_ref[...]
    wa = wattn_ref[...]
    logits = lax.dot_general(x, wa, (((1,), (1,)), ((), ())),
                             preferred_element_type=jnp.float32)
    m0 = jnp.max(logits[0:L, :])
    m1 = jnp.max(logits[L:2 * L, :])
    rid1 = lax.broadcasted_iota(jnp.int32, (B * L, 1), 0)
    m = jnp.where(rid1 < L, m0, m1)
    ev = jnp.exp(logits - m)
    xw = x * ev
    ri = lax.broadcasted_iota(jnp.int32, (B * L, B * L), 0)
    ci = lax.broadcasted_iota(jnp.int32, (B * L, B * L), 1)
    tri = jnp.where((ci <= ri) & ((ri // L) == (ci // L)), 1.0, 0.0)
    cinc = lax.dot_general(tri, xw, (((1,), (0,)), ((), ())),
                           preferred_element_type=jnp.float32)
    zinc = lax.dot_general(tri, ev, (((1,), (0,)), ((), ())),
                           preferred_element_type=jnp.float32)
    c_ref[0:B * L, :] = cinc
    c_ref[B * L:ROWS, :] = cinc - xw
    z_ref[0:B * L, :] = zinc
    z_ref[B * L:ROWS, :] = zinc - ev


_NC, _NS = 2, 16
_NW = _NC * _NS
_QH = Q // _NW
_QPW = 2 * _QH
_CH = _QPW // 16


def _sc_gather(ctab, ztab, qb, s1, e1, s2, e2):
    mesh = plsc.VectorSubcoreMesh(core_axis_name="c", subcore_axis_name="s")

    @functools.partial(
        pl.kernel,
        out_type=[jax.ShapeDtypeStruct((Q, P), jnp.float32),
                  jax.ShapeDtypeStruct((Q, P), jnp.float32)],
        mesh=mesh,
        compiler_params=pltpu.CompilerParams(needs_layout_passes=False),
        scratch_types=[
            pltpu.VMEM((_QPW,), jnp.int32),
            pltpu.VMEM((_QPW,), jnp.int32),
            pltpu.VMEM((_QPW,), jnp.int32),
            pltpu.VMEM((_QPW,), jnp.int32),
            pltpu.VMEM((_QPW,), jnp.int32),
            pltpu.VMEM((ROWS,), jnp.float32),
            pltpu.VMEM((_QPW,), jnp.float32),
            pltpu.VMEM((_QPW, P), jnp.float32),
            pltpu.VMEM((_QPW, P), jnp.float32),
            pltpu.VMEM((_QPW, P), jnp.float32),
            pltpu.SemaphoreType.DMA,
            pltpu.SemaphoreType.DMA,
        ],
    )
    def k(ctab_hbm, ztab_hbm, qb_hbm, s1_hbm, e1_hbm, s2_hbm, e2_hbm,
          out1_hbm, out2_hbm,
          qb_v, s_v, e_v, ie_v, is_v, z_v, rec_v, bufe, bufs, obuf,
          sem_e, sem_s):
        wid = lax.axis_index("s") * _NC + lax.axis_index("c")
        base = wid * _QH

        pltpu.sync_copy(qb_hbm.at[pl.ds(base, _QH)], qb_v.at[pl.ds(0, _QH)])
        pltpu.sync_copy(qb_hbm.at[pl.ds(base, _QH)], qb_v.at[pl.ds(_QH, _QH)])
        pltpu.sync_copy(s1_hbm.at[pl.ds(base, _QH)], s_v.at[pl.ds(0, _QH)])
        pltpu.sync_copy(s2_hbm.at[pl.ds(base, _QH)], s_v.at[pl.ds(_QH, _QH)])
        pltpu.sync_copy(e1_hbm.at[pl.ds(base, _QH)], e_v.at[pl.ds(0, _QH)])
        pltpu.sync_copy(e2_hbm.at[pl.ds(base, _QH)], e_v.at[pl.ds(_QH, _QH)])

        for c in range(_CH):
            sl = pl.ds(c * 16, 16)
            row = qb_v[sl] * L
            ie_v[sl] = row + e_v[sl]
            is_v[sl] = row + s_v[sl] + (B * L)
        cp_e = pltpu.async_copy(ctab_hbm.at[ie_v], bufe, sem_e)
        cp_s = pltpu.async_copy(ctab_hbm.at[is_v], bufs, sem_s)
        pltpu.sync_copy(ztab_hbm, z_v)
        for c in range(_CH):
            sl = pl.ds(c * 16, 16)
            ze = plsc.load_gather(z_v, [ie_v[sl]])
            zs = plsc.load_gather(z_v, [is_v[sl]])
            valid = s_v[sl] <= e_v[sl]
            rec_v[sl] = jnp.where(valid, 1.0 / (ze - zs), 0.0)
        cp_e.wait()
        cp_s.wait()

        def qbody(q, carry):
            r = plsc.load_gather(rec_v, [jnp.zeros((16,), jnp.int32) + q])
            for f in range(P // 16):
                fsl = pl.ds(f * 16, 16)
                obuf[q, fsl] = (bufe[q, fsl] - bufs[q, fsl]) * r
            return carry

        lax.fori_loop(0, _QPW, qbody, 0)

        pltpu.sync_copy(obuf.at[pl.ds(0, _QH)], out1_hbm.at[pl.ds(base, _QH)])
        pltpu.sync_copy(obuf.at[pl.ds(_QH, _QH)], out2_hbm.at[pl.ds(base, _QH)])

    return k(ctab, ztab, qb, s1, e1, s2, e2)


def kernel(flag, encoded_input, start_ids_1, end_ids_1, query_batch_idx,
           start_ids_2, end_ids_2, W_proj, b_proj, w_attn, b_attn):
    enc2 = encoded_input.reshape(B * L, D_IN).astype(jnp.float32)
    ctab, ztab2 = pl.pallas_call(
        _tc_tables,
        out_shape=[
            jax.ShapeDtypeStruct((ROWS, P), jnp.float32),
            jax.ShapeDtypeStruct((ROWS, 1), jnp.float32),
        ],
    )(enc2, W_proj.astype(jnp.float32),
      b_proj.reshape(1, P).astype(jnp.float32),
      w_attn.reshape(1, P).astype(jnp.float32))
    res1, res2 = _sc_gather(
        ctab, ztab2.reshape(ROWS),
        query_batch_idx.astype(jnp.int32),
        start_ids_1.astype(jnp.int32), end_ids_1.astype(jnp.int32),
        start_ids_2.astype(jnp.int32), end_ids_2.astype(jnp.int32))
    return res1, res2


# speedup vs baseline: 1.1089x; 1.1089x over previous
import functools

import jax
import jax.numpy as jnp
from jax import lax
from jax.experimental import pallas as pl
from jax.experimental.pallas import tpu as pltpu
from jax.experimental.pallas import tpu_sc as plsc

B = 2
L = 256
D_IN = 768
P = 256
Q = 1024
BL = B * L
ROWS = 2 * BL


def _tc_tables(enc_ref, w_ref, bproj_ref, wattn_ref, c_ref, z_ref):
    enc = enc_ref[...]
    w = w_ref[...]
    x = lax.dot_general(enc, w, (((1,), (1,)), ((), ())),
                        preferred_element_type=jnp.float32)
    x = x + bproj_ref[...]
    wa = wattn_ref[...]
    lrow = lax.dot_general(wa, x, (((1,), (1,)), ((), ())),
                           preferred_element_type=jnp.float32)
    m0 = jnp.max(lrow[:, 0:L])
    m1 = jnp.max(lrow[:, L:BL])
    ci1 = lax.broadcasted_iota(jnp.int32, (1, BL), 1)
    m = jnp.where(ci1 < L, m0, m1)
    evr = jnp.exp(lrow - m)
    ri = lax.broadcasted_iota(jnp.int32, (BL, BL), 0)
    ci = lax.broadcasted_iota(jnp.int32, (BL, BL), 1)
    same = (ri // L) == (ci // L)
    tri_inc = jnp.where((ci <= ri) & same, 1.0, 0.0)
    tri_exc = jnp.where((ci < ri) & same, 1.0, 0.0)
    c_ref[0:BL, :] = lax.dot_general(
        tri_inc * evr, x, (((1,), (0,)), ((), ())),
        preferred_element_type=jnp.float32)
    c_ref[BL:ROWS, :] = lax.dot_general(
        tri_exc * evr, x, (((1,), (0,)), ((), ())),
        preferred_element_type=jnp.float32)
    zrow = lax.dot_general(evr, tri_inc, (((1,), (1,)), ((), ())),
                           preferred_element_type=jnp.float32)
    z_ref[0:1, :] = zrow
    z_ref[1:2, :] = zrow - evr


_NC, _NS = 2, 16
_NW = _NC * _NS
_QH = Q // _NW
_QPW = 2 * _QH
_CH = _QPW // 16


def _sc_gather(ctab, ztab, qb, s1, e1, s2, e2):
    mesh = plsc.VectorSubcoreMesh(core_axis_name="c", subcore_axis_name="s")

    @functools.partial(
        pl.kernel,
        out_type=[jax.ShapeDtypeStruct((Q, P), jnp.float32),
                  jax.ShapeDtypeStruct((Q, P), jnp.float32)],
        mesh=mesh,
        compiler_params=pltpu.CompilerParams(needs_layout_passes=False),
        scratch_types=[
            pltpu.VMEM((_QPW,), jnp.int32),
            pltpu.VMEM((_QPW,), jnp.int32),
            pltpu.VMEM((_QPW,), jnp.int32),
            pltpu.VMEM((_QPW,), jnp.int32),
            pltpu.VMEM((_QPW,), jnp.int32),
            pltpu.VMEM((2, BL), jnp.float32),
            pltpu.VMEM((_QPW,), jnp.float32),
            pltpu.VMEM((_QPW, P), jnp.float32),
            pltpu.VMEM((_QPW, P), jnp.float32),
            pltpu.VMEM((_QPW, P), jnp.float32),
            pltpu.SemaphoreType.DMA,
            pltpu.SemaphoreType.DMA,
            pltpu.SemaphoreType.DMA,
        ],
    )
    def k(ctab_hbm, ztab_hbm, qb_hbm, s1_hbm, e1_hbm, s2_hbm, e2_hbm,
          out1_hbm, out2_hbm,
          qb_v, s_v, e_v, ie_v, is_v, z_v, rec_v, bufe, bufs, obuf,
          sem_e, sem_s, sem_i):
        wid = lax.axis_index("s") * _NC + lax.axis_index("c")
        base = wid * _QH

        cz = pltpu.async_copy(ztab_hbm, z_v, sem_i)
        c0 = pltpu.async_copy(qb_hbm.at[pl.ds(base, _QH)],
                              qb_v.at[pl.ds(0, _QH)], sem_i)
        c1 = pltpu.async_copy(qb_hbm.at[pl.ds(base, _QH)],
                              qb_v.at[pl.ds(_QH, _QH)], sem_i)
        c2 = pltpu.async_copy(s1_hbm.at[pl.ds(base, _QH)],
                              s_v.at[pl.ds(0, _QH)], sem_i)
        c3 = pltpu.async_copy(s2_hbm.at[pl.ds(base, _QH)],
                              s_v.at[pl.ds(_QH, _QH)], sem_i)
        c4 = pltpu.async_copy(e1_hbm.at[pl.ds(base, _QH)],
                              e_v.at[pl.ds(0, _QH)], sem_i)
        c5 = pltpu.async_copy(e2_hbm.at[pl.ds(base, _QH)],
                              e_v.at[pl.ds(_QH, _QH)], sem_i)
        for cp in (cz, c0, c1, c2, c3, c4, c5):
            cp.wait()

        for c in range(_CH):
            sl = pl.ds(c * 16, 16)
            row = qb_v[sl] * L
            ie_v[sl] = row + e_v[sl]
            is_v[sl] = row + s_v[sl] + BL
        cp_e = pltpu.async_copy(ctab_hbm.at[ie_v], bufe, sem_e)
        cp_s = pltpu.async_copy(ctab_hbm.at[is_v], bufs, sem_s)

        t0 = jnp.zeros((16,), jnp.int32)
        t1 = t0 + 1
        for c in range(_CH):
            sl = pl.ds(c * 16, 16)
            ze = plsc.load_gather(z_v, [t0, ie_v[sl]])
            zs = plsc.load_gather(z_v, [t1, is_v[sl] - BL])
            valid = s_v[sl] <= e_v[sl]
            rec_v[sl] = jnp.where(valid, 1.0 / (ze - zs), 0.0)
        cp_e.wait()
        cp_s.wait()

        def qbody(q, carry):
            r = plsc.load_gather(rec_v, [jnp.zeros((16,), jnp.int32) + q])
            for f in range(P // 16):
                fsl = pl.ds(f * 16, 16)
                obuf[q, fsl] = (bufe[q, fsl] - bufs[q, fsl]) * r
            return carry

        lax.fori_loop(0, _QPW, qbody, 0)

        pltpu.sync_copy(obuf.at[pl.ds(0, _QH)], out1_hbm.at[pl.ds(base, _QH)])
        pltpu.sync_copy(obuf.at[pl.ds(_QH, _QH)], out2_hbm.at[pl.ds(base, _QH)])

    return k(ctab, ztab, qb, s1, e1, s2, e2)


def kernel(flag, encoded_input, start_ids_1, end_ids_1, query_batch_idx,
           start_ids_2, end_ids_2, W_proj, b_proj, w_attn, b_attn):
    enc2 = encoded_input.reshape(BL, D_IN).astype(jnp.float32)
    ctab, ztab = pl.pallas_call(
        _tc_tables,
        out_shape=[
            jax.ShapeDtypeStruct((ROWS, P), jnp.float32),
            jax.ShapeDtypeStruct((2, BL), jnp.float32),
        ],
    )(enc2, W_proj.astype(jnp.float32),
      b_proj.reshape(1, P).astype(jnp.float32),
      w_attn.reshape(1, P).astype(jnp.float32))
    res1, res2 = _sc_gather(
        ctab, ztab,
        query_batch_idx.astype(jnp.int32),
        start_ids_1.astype(jnp.int32), end_ids_1.astype(jnp.int32),
        start_ids_2.astype(jnp.int32), end_ids_2.astype(jnp.int32))
    return res1, res2


# speedup vs baseline: 1.1189x; 1.0090x over previous
import functools

import jax
import jax.numpy as jnp
from jax import lax
from jax.experimental import pallas as pl
from jax.experimental.pallas import tpu as pltpu
from jax.experimental.pallas import tpu_sc as plsc

B = 2
L = 256
D_IN = 768
P = 256
Q = 1024
BL = B * L
ROWS = 2 * BL


def _tc_tables(enc_ref, w_ref, bproj_ref, wattn_ref, c_ref, z_ref):
    enc = enc_ref[...]
    w = w_ref[...]
    x = lax.dot_general(enc, w, (((1,), (1,)), ((), ())),
                        preferred_element_type=jnp.float32)
    x = x + bproj_ref[...]
    wa = wattn_ref[...]
    lrow = lax.dot_general(wa, x, (((1,), (1,)), ((), ())),
                           preferred_element_type=jnp.float32)
    m0 = jnp.max(lrow[:, 0:L])
    m1 = jnp.max(lrow[:, L:BL])
    ci1 = lax.broadcasted_iota(jnp.int32, (1, BL), 1)
    m = jnp.where(ci1 < L, m0, m1)
    evr = jnp.exp(lrow - m)
    ri = lax.broadcasted_iota(jnp.int32, (BL, BL), 0)
    ci = lax.broadcasted_iota(jnp.int32, (BL, BL), 1)
    same = (ri // L) == (ci // L)
    tri_inc = jnp.where((ci <= ri) & same, 1.0, 0.0)
    tri_exc = jnp.where((ci < ri) & same, 1.0, 0.0)
    c_ref[0:BL, :] = lax.dot_general(
        tri_inc * evr, x, (((1,), (0,)), ((), ())),
        preferred_element_type=jnp.float32)
    c_ref[BL:ROWS, :] = lax.dot_general(
        tri_exc * evr, x, (((1,), (0,)), ((), ())),
        preferred_element_type=jnp.float32)
    zrow = lax.dot_general(evr, tri_inc, (((1,), (1,)), ((), ())),
                           preferred_element_type=jnp.float32)
    z_ref[0:1, :] = zrow
    z_ref[1:2, :] = zrow - evr


_NC, _NS = 2, 16
_NW = _NC * _NS
_QH = Q // _NW
_QPW = 2 * _QH
_CH = _QPW // 16


def _sc_gather(ctab, ztab, qb, s1, e1, s2, e2):
    mesh = plsc.VectorSubcoreMesh(core_axis_name="c", subcore_axis_name="s")

    @functools.partial(
        pl.kernel,
        out_type=[jax.ShapeDtypeStruct((Q, P), jnp.float32),
                  jax.ShapeDtypeStruct((Q, P), jnp.float32)],
        mesh=mesh,
        compiler_params=pltpu.CompilerParams(needs_layout_passes=False),
        scratch_types=[
            pltpu.VMEM((_QPW,), jnp.int32),
            pltpu.VMEM((_QPW,), jnp.int32),
            pltpu.VMEM((_QPW,), jnp.int32),
            pltpu.VMEM((_QPW,), jnp.int32),
            pltpu.VMEM((_QPW,), jnp.int32),
            pltpu.VMEM((2, BL), jnp.float32),
            pltpu.VMEM((_QPW,), jnp.float32),
            pltpu.VMEM((_QPW, P), jnp.float32),
            pltpu.VMEM((_QPW, P), jnp.float32),
            pltpu.VMEM((_QPW, P), jnp.float32),
            pltpu.SemaphoreType.DMA,
            pltpu.SemaphoreType.DMA,
            pltpu.SemaphoreType.DMA,
            pltpu.SemaphoreType.DMA,
        ],
    )
    def k(ctab_hbm, ztab_hbm, qb_hbm, s1_hbm, e1_hbm, s2_hbm, e2_hbm,
          out1_hbm, out2_hbm,
          qb_v, s_v, e_v, ie_v, is_v, z_v, rec_v, bufe, bufs, obuf,
          sem_a, sem_b, sem_i, sem_o):
        wid = lax.axis_index("s") * _NC + lax.axis_index("c")
        base = wid * _QH

        cz = pltpu.async_copy(ztab_hbm, z_v, sem_i)
        c0 = pltpu.async_copy(qb_hbm.at[pl.ds(base, _QH)],
                              qb_v.at[pl.ds(0, _QH)], sem_i)
        c1 = pltpu.async_copy(qb_hbm.at[pl.ds(base, _QH)],
                              qb_v.at[pl.ds(_QH, _QH)], sem_i)
        c2 = pltpu.async_copy(s1_hbm.at[pl.ds(base, _QH)],
                              s_v.at[pl.ds(0, _QH)], sem_i)
        c3 = pltpu.async_copy(s2_hbm.at[pl.ds(base, _QH)],
                              s_v.at[pl.ds(_QH, _QH)], sem_i)
        c4 = pltpu.async_copy(e1_hbm.at[pl.ds(base, _QH)],
                              e_v.at[pl.ds(0, _QH)], sem_i)
        c5 = pltpu.async_copy(e2_hbm.at[pl.ds(base, _QH)],
                              e_v.at[pl.ds(_QH, _QH)], sem_i)
        for cp in (c0, c1, c2, c3, c4, c5):
            cp.wait()

        for c in range(_CH):
            sl = pl.ds(c * 16, 16)
            row = qb_v[sl] * L
            ie_v[sl] = row + e_v[sl]
            is_v[sl] = row + s_v[sl] + BL

        h0 = pl.ds(0, _QH)
        h1 = pl.ds(_QH, _QH)
        ge0 = pltpu.async_copy(ctab_hbm.at[ie_v.at[h0]], bufe.at[h0], sem_a)
        gs0 = pltpu.async_copy(ctab_hbm.at[is_v.at[h0]], bufs.at[h0], sem_a)
        ge1 = pltpu.async_copy(ctab_hbm.at[ie_v.at[h1]], bufe.at[h1], sem_b)
        gs1 = pltpu.async_copy(ctab_hbm.at[is_v.at[h1]], bufs.at[h1], sem_b)

        cz.wait()
        t0 = jnp.zeros((16,), jnp.int32)
        t1 = t0 + 1
        for c in range(_CH):
            sl = pl.ds(c * 16, 16)
            ze = plsc.load_gather(z_v, [t0, ie_v[sl]])
            zs = plsc.load_gather(z_v, [t1, is_v[sl] - BL])
            valid = s_v[sl] <= e_v[sl]
            rec_v[sl] = jnp.where(valid, 1.0 / (ze - zs), 0.0)

        def qbody(q, carry):
            r = plsc.load_gather(rec_v, [jnp.zeros((16,), jnp.int32) + q])
            for f in range(P // 16):
                fsl = pl.ds(f * 16, 16)
                obuf[q, fsl] = (bufe[q, fsl] - bufs[q, fsl]) * r
            return carry

        ge0.wait()
        gs0.wait()
        lax.fori_loop(0, _QH, qbody, 0)
        o1 = pltpu.async_copy(obuf.at[h0], out1_hbm.at[pl.ds(base, _QH)],
                              sem_o)
        ge1.wait()
        gs1.wait()
        lax.fori_loop(_QH, _QPW, qbody, 0)
        o2 = pltpu.async_copy(obuf.at[h1], out2_hbm.at[pl.ds(base, _QH)],
                              sem_o)
        o1.wait()
        o2.wait()

    return k(ctab, ztab, qb, s1, e1, s2, e2)


def kernel(flag, encoded_input, start_ids_1, end_ids_1, query_batch_idx,
           start_ids_2, end_ids_2, W_proj, b_proj, w_attn, b_attn):
    enc2 = encoded_input.reshape(BL, D_IN).astype(jnp.float32)
    ctab, ztab = pl.pallas_call(
        _tc_tables,
        out_shape=[
            jax.ShapeDtypeStruct((ROWS, P), jnp.float32),
            jax.ShapeDtypeStruct((2, BL), jnp.float32),
        ],
    )(enc2, W_proj.astype(jnp.float32),
      b_proj.reshape(1, P).astype(jnp.float32),
      w_attn.reshape(1, P).astype(jnp.float32))
    res1, res2 = _sc_gather(
        ctab, ztab,
        query_batch_idx.astype(jnp.int32),
        start_ids_1.astype(jnp.int32), end_ids_1.astype(jnp.int32),
        start_ids_2.astype(jnp.int32), end_ids_2.astype(jnp.int32))
    return res1, res2
